# Initial kernel scaffold; baseline (speedup 1.0000x reference)
import jax
import jax.numpy as jnp
from jax.experimental import pallas as pl


def kernel(x, pos_emb):
    raise NotImplementedError("write your pallas kernel here")


# speedup vs baseline: 7.3592x; 7.3592x over previous
import jax
import jax.numpy as jnp
from jax.experimental import pallas as pl


def _body(pe_ref, o_ref):
    o_ref[...] = jnp.broadcast_to(pe_ref[...][None, :, :], o_ref.shape)


def kernel(x, pos_emb):
    B, T = x.shape
    D = pos_emb.shape[1]
    pe = pos_emb[:T]
    BLK = 32
    return pl.pallas_call(
        _body,
        grid=(B // BLK,),
        in_specs=[pl.BlockSpec((T, D), lambda i: (0, 0))],
        out_specs=pl.BlockSpec((BLK, T, D), lambda i: (i, 0, 0)),
        out_shape=jax.ShapeDtypeStruct((B, T, D), pos_emb.dtype),
    )(pe)


# speedup vs baseline: 12.1517x; 1.6512x over previous
import jax
import jax.numpy as jnp
from jax.experimental import pallas as pl


def _body(pe_ref, o_ref):
    o_ref[...] = jnp.broadcast_to(pe_ref[...], o_ref.shape)


def kernel(x, pos_emb):
    B, T = x.shape
    D = pos_emb.shape[1]
    pe = pos_emb[:T].reshape(1, T * D)
    BLK = 128
    out = pl.pallas_call(
        _body,
        grid=(B // BLK,),
        in_specs=[pl.BlockSpec((1, T * D), lambda i: (0, 0))],
        out_specs=pl.BlockSpec((BLK, T * D), lambda i: (i, 0)),
        out_shape=jax.ShapeDtypeStruct((B, T * D), pos_emb.dtype),
    )(pe)
    return out.reshape(B, T, D)
